# Initial kernel scaffold; baseline (speedup 1.0000x reference)
import jax
import jax.numpy as jnp
from jax.experimental import pallas as pl


def kernel(Q, K, V, miu, W_Q, b_Q, W_K, b_K):
    raise NotImplementedError("write your pallas kernel here")


# speedup vs baseline: 5.5402x; 5.5402x over previous
import functools
import math

import jax
import jax.numpy as jnp
from jax.experimental import pallas as pl


def _gelu(y):
    return 0.5 * y * (1.0 + jax.lax.erf(y * (1.0 / math.sqrt(2.0))))


def _route_body(x_ref, miuT_ref, lane_ref, sub_ref):
    x = x_ref[0]
    s = jnp.dot(x, miuT_ref[...], preferred_element_type=jnp.float32)
    mx = jnp.max(s, axis=1, keepdims=True)
    m_n = s.shape[1]
    iota = jax.lax.broadcasted_iota(jnp.int32, s.shape, 1)
    idx2 = jnp.min(jnp.where(s >= mx, iota, m_n), axis=1, keepdims=True)
    sub_ref[0] = idx2
    lane_ref[0, 0, :] = idx2[:, 0]


def _transform_body(x_ref, w_ref, b_ref, a_ref, out_ref):
    m = pl.program_id(1)
    x = x_ref[0]
    y = jnp.dot(x, w_ref[0], preferred_element_type=jnp.float32) + b_ref[0]
    y = _gelu(y)
    sel = a_ref[0] == m

    @pl.when(m == 0)
    def _():
        out_ref[0] = jnp.where(sel, y, 0.0)

    @pl.when(m > 0)
    def _():
        out_ref[0] = jnp.where(sel, y, out_ref[0])


def _attn_body(q_ref, k_ref, aq_ref, ak_ref, out_ref, *, scale):
    q = q_ref[0]
    k = k_ref[0]
    logits = jax.lax.dot_general(
        q, k, (((1,), (1,)), ((), ())), preferred_element_type=jnp.float32
    ) * scale
    aq = aq_ref[0]
    ak = ak_ref[0]
    same = aq == ak
    att = jnp.where(same, logits, -1e9)
    mx = jnp.max(att, axis=1, keepdims=True)
    p = jnp.exp(att - mx)
    l = jnp.sum(p, axis=1, keepdims=True)
    o = jnp.dot(p, k, preferred_element_type=jnp.float32) / l
    has = jnp.any(same, axis=1, keepdims=True)
    out_ref[0] = jnp.where(has, o, 0.0)


def kernel(Q, K, V, miu, W_Q, b_Q, W_K, b_K):
    B, SQ, D = Q.shape
    SK = K.shape[1]
    M = miu.shape[0]
    del V

    X4 = jnp.concatenate([Q, K], axis=0)

    a_lane, a_sub = pl.pallas_call(
        _route_body,
        grid=(2 * B,),
        in_specs=[
            pl.BlockSpec((1, SQ, D), lambda p: (p, 0, 0)),
            pl.BlockSpec((D, M), lambda p: (0, 0)),
        ],
        out_specs=[
            pl.BlockSpec((1, 1, SQ), lambda p: (p, 0, 0)),
            pl.BlockSpec((1, SQ, 1), lambda p: (p, 0, 0)),
        ],
        out_shape=[
            jax.ShapeDtypeStruct((2 * B, 1, SQ), jnp.int32),
            jax.ShapeDtypeStruct((2 * B, SQ, 1), jnp.int32),
        ],
    )(X4, miu.T)

    Ws = jnp.concatenate([W_Q, W_K], axis=0)
    bs = jnp.concatenate([b_Q, b_K], axis=0).reshape(2 * M, 1, D)

    xt = pl.pallas_call(
        _transform_body,
        grid=(2 * B, M),
        in_specs=[
            pl.BlockSpec((1, SQ, D), lambda p, m: (p, 0, 0)),
            pl.BlockSpec((1, D, D), lambda p, m: (m + M * (p // B), 0, 0)),
            pl.BlockSpec((1, 1, D), lambda p, m: (m + M * (p // B), 0, 0)),
            pl.BlockSpec((1, SQ, 1), lambda p, m: (p, 0, 0)),
        ],
        out_specs=pl.BlockSpec((1, SQ, D), lambda p, m: (p, 0, 0)),
        out_shape=jax.ShapeDtypeStruct((2 * B, SQ, D), jnp.float32),
    )(X4, Ws, bs, a_sub)

    BQ = 512
    O = pl.pallas_call(
        functools.partial(_attn_body, scale=1.0 / math.sqrt(D)),
        grid=(B, SQ // BQ),
        in_specs=[
            pl.BlockSpec((1, BQ, D), lambda b, j: (b, j, 0)),
            pl.BlockSpec((1, SK, D), lambda b, j: (B + b, 0, 0)),
            pl.BlockSpec((1, BQ, 1), lambda b, j: (b, j, 0)),
            pl.BlockSpec((1, 1, SK), lambda b, j: (B + b, 0, 0)),
        ],
        out_specs=pl.BlockSpec((1, BQ, D), lambda b, j: (b, j, 0)),
        out_shape=jax.ShapeDtypeStruct((B, SQ, D), jnp.float32),
    )(xt, xt, a_sub, a_lane)

    return O
